# Initial kernel scaffold; baseline (speedup 1.0000x reference)
import jax
import jax.numpy as jnp
from jax.experimental import pallas as pl


def kernel(X, A, W0, b0, Wd0, bd0, Wd1, bd1, W1, b1):
    raise NotImplementedError("write your pallas kernel here")


# speedup vs baseline: 1.2430x; 1.2430x over previous
import functools

import jax
import jax.numpy as jnp
from jax import lax
from jax.experimental import pallas as pl
from jax.experimental.pallas import tpu as pltpu
from jax.experimental.pallas import tpu_sc as plsc

_N = 512
_D = 64
_E = 16384
_OUT = 64

_NW = 32
_ROWS_PER_W = _N // _NW
_BR = 8
_NB = _N // _BR


def _adj_body(row_hbm, col_hbm, adj_hbm, row_v, col_v, slab_v):
    c = lax.axis_index("c")
    s = lax.axis_index("s")
    wid = s * 2 + c
    base = wid * _ROWS_PER_W

    pltpu.sync_copy(row_hbm, row_v)
    pltpu.sync_copy(col_hbm, col_v)

    def zero_row(i, carry):
        r = i // (_N // 16)
        c16 = (i % (_N // 16)) * 16
        slab_v[r, pl.ds(c16, 16)] = jnp.zeros((16,), jnp.float32)
        return carry

    lax.fori_loop(0, _ROWS_PER_W * (_N // 16), zero_row, 0)

    def edge(i, carry):
        d = row_v[pl.ds(i * 16, 16)]
        sv = col_v[pl.ds(i * 16, 16)]
        inr = (d >= base) & (d < base + _ROWS_PER_W)
        r = jnp.where(inr, d - base, 0)
        cidx = jnp.where(inr, sv, 0)
        flat = r * _N + cidx
        cnt, last = plsc.scan_count(flat, mask=inr)
        plsc.addupdate_scatter(
            slab_v, [r, cidx], cnt.astype(jnp.float32), mask=last & inr)
        return carry

    lax.fori_loop(0, _E // 16, edge, 0)

    pltpu.sync_copy(slab_v, adj_hbm.at[pl.ds(base, _ROWS_PER_W)])


def _build_adj(row_idx, col_idx):
    mesh = plsc.VectorSubcoreMesh(core_axis_name="c", subcore_axis_name="s")
    fn = functools.partial(
        pl.kernel,
        out_type=jax.ShapeDtypeStruct((_N, _N), jnp.float32),
        mesh=mesh,
        scratch_types=[
            pltpu.VMEM((_E,), jnp.int32),
            pltpu.VMEM((_E,), jnp.int32),
            pltpu.VMEM((_ROWS_PER_W, _N), jnp.float32),
        ],
        compiler_params=pltpu.CompilerParams(needs_layout_passes=False),
    )(_adj_body)
    return fn(row_idx, col_idx)


def _stats_body(x_ref, w0_ref, b0_ref, xsum_ref, xlsum_ref, xdiag_ref,
                xps_ref):
    ib = pl.program_id(0)
    xb = x_ref[...]
    xf = xb.reshape(_BR * _N, _D)
    xl = jnp.maximum(xf @ w0_ref[...] + b0_ref[...], 0.0)

    @pl.when(ib == 0)
    def _():
        xsum_ref[...] = jnp.zeros_like(xsum_ref)
        xlsum_ref[...] = jnp.zeros_like(xlsum_ref)

    xsum_ref[...] += xb.sum(axis=0)
    xlsum_ref[...] += xl.reshape(_BR, _N, _D).sum(axis=0)
    xps_ref[...] = xb.mean(axis=1)
    for b in range(_BR):
        i = ib * _BR + b
        xdiag_ref[b, :] = x_ref[b, pl.ds(i, 1), :].reshape(_D)


def _run_stats(x, w0, b0):
    grid = (_NB,)
    return pl.pallas_call(
        _stats_body,
        grid=grid,
        in_specs=[
            pl.BlockSpec((_BR, _N, _D), lambda i: (i, 0, 0)),
            pl.BlockSpec((_D, _D), lambda i: (0, 0)),
            pl.BlockSpec((1, _D), lambda i: (0, 0)),
        ],
        out_specs=[
            pl.BlockSpec((_N, _D), lambda i: (0, 0)),
            pl.BlockSpec((_N, _D), lambda i: (0, 0)),
            pl.BlockSpec((_BR, _D), lambda i: (i, 0)),
            pl.BlockSpec((_BR, _D), lambda i: (i, 0)),
        ],
        out_shape=[
            jax.ShapeDtypeStruct((_N, _D), jnp.float32),
            jax.ShapeDtypeStruct((_N, _D), jnp.float32),
            jax.ShapeDtypeStruct((_N, _D), jnp.float32),
            jax.ShapeDtypeStruct((_N, _D), jnp.float32),
        ],
    )(x, w0, b0.reshape(1, _D))


def _main_body(x_ref, adj_ref, w0_ref, b0_ref, aall_ref, bpack_ref,
               xsum_ref, xlsum_ref, xdiag_ref, xps_ref, bvec_ref,
               bdiag_ref, o_ref, rmat_ref, cmat_ref, dvec_ref):
    ib = pl.program_id(0)

    @pl.when(ib == 0)
    def _():
        inv_n = 1.0 / _N
        xpn = xsum_ref[...] * inv_n
        xlpn = xlsum_ref[...] * inv_n
        xdiag = xdiag_ref[...]
        xps = xps_ref[...]
        x4pn = adj_ref[...] @ xlpn
        a2 = aall_ref[64:128, :]
        a3 = aall_ref[128:192, :]
        a5 = aall_ref[256:320, :]
        a6 = aall_ref[320:384, :]
        a7 = aall_ref[384:448, :]
        b123 = bpack_ref[0:64, :]
        b5 = bpack_ref[128:192, :]
        b6 = bpack_ref[192:256, :]
        b7 = bpack_ref[256:320, :]
        rmat_ref[...] = xdiag @ a3 + xpn @ a5 + x4pn @ a7
        cmat_ref[...] = xdiag @ a2 + xps @ a6 + bvec_ref[...]
        dvec_ref[...] = (xdiag @ b123 + xpn @ b5 + xps @ b6 + x4pn @ b7
                         + bdiag_ref[...])

    a1 = aall_ref[0:64, :]
    a4 = aall_ref[192:256, :]
    b4 = bpack_ref[64:128, :]
    w0 = w0_ref[...]
    b0 = b0_ref[...]
    adj = adj_ref[...]
    cmat = cmat_ref[...]

    for b in range(_BR):
        i = ib * _BR + b
        xb = x_ref[b]
        xl = jnp.maximum(xb @ w0 + b0, 0.0)
        x4 = adj @ xl
        rrow = rmat_ref[pl.ds(i, 1), :]
        o = xb @ a1 + x4 @ a4 + rrow + cmat
        o_ref[b] = jnp.maximum(o, 0.0)
        adj_row = adj_ref[pl.ds(i, 1), :]
        od = dvec_ref[pl.ds(i, 1), :] + (adj_row @ xl) @ b4
        o_ref[b, pl.ds(i, 1), :] = jnp.maximum(od, 0.0)


def _run_main(x, adj, w0, b0, aall, bpack, xsum, xlsum, xdiag, xps, bvec,
              bdiag):
    grid = (_NB,)
    const = lambda i: (0, 0)
    return pl.pallas_call(
        _main_body,
        grid=grid,
        in_specs=[
            pl.BlockSpec((_BR, _N, _D), lambda i: (i, 0, 0)),
            pl.BlockSpec((_N, _N), const),
            pl.BlockSpec((_D, _D), const),
            pl.BlockSpec((1, _D), const),
            pl.BlockSpec((448, _OUT), const),
            pl.BlockSpec((320, _OUT), const),
            pl.BlockSpec((_N, _D), const),
            pl.BlockSpec((_N, _D), const),
            pl.BlockSpec((_N, _D), const),
            pl.BlockSpec((_N, _D), const),
            pl.BlockSpec((1, _OUT), const),
            pl.BlockSpec((1, _OUT), const),
        ],
        out_specs=pl.BlockSpec((_BR, _N, _OUT), lambda i: (i, 0, 0)),
        out_shape=jax.ShapeDtypeStruct((_N, _N, _OUT), jnp.float32),
        scratch_shapes=[
            pltpu.VMEM((_N, _OUT), jnp.float32),
            pltpu.VMEM((_N, _OUT), jnp.float32),
            pltpu.VMEM((_N, _OUT), jnp.float32),
        ],
    )(x, adj, w0, b0.reshape(1, _D), aall, bpack, xsum, xlsum, xdiag, xps,
      bvec.reshape(1, _OUT), bdiag.reshape(1, _OUT))


@jax.jit
def kernel(X, A, W0, b0, Wd0, bd0, Wd1, bd1, W1, b1):
    aall = Wd0 @ W1
    ball = Wd1 @ W1
    bvec = bd0 @ W1 + b1
    bdiag = bd1 @ W1 + b1
    b123 = ball[0:64] + ball[64:128] + ball[128:192]
    bpack = jnp.concatenate(
        [b123, ball[192:256], ball[256:320], ball[320:384], ball[384:448]],
        axis=0)

    adj = _build_adj(A[1], A[0])
    xsum, xlsum, xdiag, xps = _run_stats(X, W0, b0)
    return _run_main(X, adj, W0, b0, aall, bpack, xsum, xlsum, xdiag, xps,
                     bvec, bdiag)


# speedup vs baseline: 1.9035x; 1.5314x over previous
import functools

import jax
import jax.numpy as jnp
from jax import lax
from jax.experimental import pallas as pl
from jax.experimental.pallas import tpu as pltpu
from jax.experimental.pallas import tpu_sc as plsc

_N = 512
_D = 64
_E = 16384
_OUT = 64

_NW = 32
_ROWS_PER_W = _N // _NW
_BR = 8
_NB = _N // _BR


def _adj_body(row_hbm, col_hbm, adj_hbm, row_v, col_v, slab_v):
    c = lax.axis_index("c")
    s = lax.axis_index("s")
    wid = s * 2 + c
    base = wid * _ROWS_PER_W

    pltpu.sync_copy(row_hbm, row_v)
    pltpu.sync_copy(col_hbm, col_v)

    def zero_row(i, carry):
        r = i // (_N // 16)
        c16 = (i % (_N // 16)) * 16
        slab_v[r, pl.ds(c16, 16)] = jnp.zeros((16,), jnp.float32)
        return carry

    lax.fori_loop(0, _ROWS_PER_W * (_N // 16), zero_row, 0)

    def edge(i, carry):
        d = row_v[pl.ds(i * 16, 16)]
        sv = col_v[pl.ds(i * 16, 16)]
        inr = (d >= base) & (d < base + _ROWS_PER_W)
        r = jnp.where(inr, d - base, 0)
        cidx = jnp.where(inr, sv, 0)
        flat = r * _N + cidx
        cnt, last = plsc.scan_count(flat, mask=inr)
        plsc.addupdate_scatter(
            slab_v, [r, cidx], cnt.astype(jnp.float32), mask=last & inr)
        return carry

    lax.fori_loop(0, _E // 16, edge, 0)

    pltpu.sync_copy(slab_v, adj_hbm.at[pl.ds(base, _ROWS_PER_W)])


def _build_adj(row_idx, col_idx):
    mesh = plsc.VectorSubcoreMesh(core_axis_name="c", subcore_axis_name="s")
    fn = functools.partial(
        pl.kernel,
        out_type=jax.ShapeDtypeStruct((_N, _N), jnp.float32),
        mesh=mesh,
        scratch_types=[
            pltpu.VMEM((_E,), jnp.int32),
            pltpu.VMEM((_E,), jnp.int32),
            pltpu.VMEM((_ROWS_PER_W, _N), jnp.float32),
        ],
        compiler_params=pltpu.CompilerParams(needs_layout_passes=False),
    )(_adj_body)
    return fn(row_idx, col_idx)


def _stats_body(x_ref, w0_ref, b0_ref, xsum_ref, xlsum_ref, xdiag_ref,
                xps_ref):
    ib = pl.program_id(0)
    xb = x_ref[...]
    xf = xb.reshape(_BR * _N, _D)
    xl = jnp.maximum(xf @ w0_ref[...] + b0_ref[...], 0.0)

    @pl.when(ib == 0)
    def _():
        xsum_ref[...] = jnp.zeros_like(xsum_ref)
        xlsum_ref[...] = jnp.zeros_like(xlsum_ref)

    xsum_ref[...] += xb.sum(axis=0)
    xlsum_ref[...] += xl.reshape(_BR, _N, _D).sum(axis=0)
    xps_ref[...] = xb.mean(axis=1)
    for b in range(_BR):
        i = ib * _BR + b
        xdiag_ref[b, :] = x_ref[b, pl.ds(i, 1), :].reshape(_D)


def _run_stats(x, w0, b0):
    grid = (_NB,)
    return pl.pallas_call(
        _stats_body,
        grid=grid,
        in_specs=[
            pl.BlockSpec((_BR, _N, _D), lambda i: (i, 0, 0)),
            pl.BlockSpec((_D, _D), lambda i: (0, 0)),
            pl.BlockSpec((1, _D), lambda i: (0, 0)),
        ],
        out_specs=[
            pl.BlockSpec((_N, _D), lambda i: (0, 0)),
            pl.BlockSpec((_N, _D), lambda i: (0, 0)),
            pl.BlockSpec((_BR, _D), lambda i: (i, 0)),
            pl.BlockSpec((_BR, _D), lambda i: (i, 0)),
        ],
        out_shape=[
            jax.ShapeDtypeStruct((_N, _D), jnp.float32),
            jax.ShapeDtypeStruct((_N, _D), jnp.float32),
            jax.ShapeDtypeStruct((_N, _D), jnp.float32),
            jax.ShapeDtypeStruct((_N, _D), jnp.float32),
        ],
    )(x, w0, b0.reshape(1, _D))


def _dot(a, b):
    return jax.lax.dot_general(a, b, (((1,), (0,)), ((), ())),
                               preferred_element_type=jnp.float32)


def _main_body(x_ref, adjb_ref, w0a1_ref, b0_ref, a4b4_ref, apack_ref,
               bpack_ref, xsum_ref, xlsum_ref, xdiag_ref, xps_ref, bvec_ref,
               bdiag_ref, o_ref, rmat_ref, cmat_ref, dvec_ref):
    ib = pl.program_id(0)

    @pl.when(ib == 0)
    def _():
        inv_n = 1.0 / _N
        xpn = xsum_ref[...] * inv_n
        xlpn = xlsum_ref[...] * inv_n
        xdiag = xdiag_ref[...]
        xps = xps_ref[...]
        x4pn = _dot(adjb_ref[...], (xlpn).astype(jnp.bfloat16))
        a2 = apack_ref[0:64, :]
        a3 = apack_ref[64:128, :]
        a5 = apack_ref[128:192, :]
        a6 = apack_ref[192:256, :]
        a7 = apack_ref[256:320, :]
        b123 = bpack_ref[0:64, :]
        b5 = bpack_ref[64:128, :]
        b6 = bpack_ref[128:192, :]
        b7 = bpack_ref[192:256, :]
        rmat_ref[...] = xdiag @ a3 + xpn @ a5 + x4pn @ a7
        cmat_ref[...] = xdiag @ a2 + xps @ a6 + bvec_ref[...]
        dvec_ref[...] = (xdiag @ b123 + xpn @ b5 + xps @ b6 + x4pn @ b7
                         + bdiag_ref[...])

    adjb = adjb_ref[...]
    cmat = cmat_ref[...]

    xf = x_ref[...].reshape(_BR * _N, _D).astype(jnp.bfloat16)
    t1 = _dot(xf, w0a1_ref[...])
    xl = jnp.maximum(t1[:, :_D] + b0_ref[...], 0.0)
    oa = t1[:, _D:].reshape(_BR, _N, _OUT)
    t2 = _dot(xl.astype(jnp.bfloat16), a4b4_ref[...])
    t2 = t2.astype(jnp.bfloat16).reshape(_BR, _N, 2 * _OUT)

    iota = jax.lax.broadcasted_iota(jnp.int32, (_N, 1), 0)
    for b in range(_BR):
        i = ib * _BR + b
        x4d = _dot(adjb, t2[b])
        rrow = rmat_ref[pl.ds(i, 1), :]
        o = oa[b] + x4d[:, :_OUT] + rrow + cmat
        od = dvec_ref[pl.ds(i, 1), :] + x4d[:, _OUT:]
        o_ref[b] = jnp.where(iota == i, jnp.maximum(od, 0.0),
                             jnp.maximum(o, 0.0))


def _run_main(x, adjb, w0a1, b0, a4b4, apack, bpack, xsum, xlsum, xdiag,
              xps, bvec, bdiag):
    grid = (_NB,)
    const = lambda i: (0, 0)
    return pl.pallas_call(
        _main_body,
        grid=grid,
        in_specs=[
            pl.BlockSpec((_BR, _N, _D), lambda i: (i, 0, 0)),
            pl.BlockSpec((_N, _N), const),
            pl.BlockSpec((_D, 2 * _D), const),
            pl.BlockSpec((1, _D), const),
            pl.BlockSpec((_D, 2 * _OUT), const),
            pl.BlockSpec((320, _OUT), const),
            pl.BlockSpec((256, _OUT), const),
            pl.BlockSpec((_N, _D), const),
            pl.BlockSpec((_N, _D), const),
            pl.BlockSpec((_N, _D), const),
            pl.BlockSpec((_N, _D), const),
            pl.BlockSpec((1, _OUT), const),
            pl.BlockSpec((1, _OUT), const),
        ],
        out_specs=pl.BlockSpec((_BR, _N, _OUT), lambda i: (i, 0, 0)),
        out_shape=jax.ShapeDtypeStruct((_N, _N, _OUT), jnp.float32),
        scratch_shapes=[
            pltpu.VMEM((_N, _OUT), jnp.float32),
            pltpu.VMEM((_N, _OUT), jnp.float32),
            pltpu.VMEM((_N, _OUT), jnp.float32),
        ],
    )(x, adjb, w0a1, b0.reshape(1, _D), a4b4, apack, bpack, xsum, xlsum,
      xdiag, xps, bvec.reshape(1, _OUT), bdiag.reshape(1, _OUT))


@jax.jit
def kernel(X, A, W0, b0, Wd0, bd0, Wd1, bd1, W1, b1):
    aall = Wd0 @ W1
    ball = Wd1 @ W1
    bvec = bd0 @ W1 + b1
    bdiag = bd1 @ W1 + b1
    b123 = ball[0:64] + ball[64:128] + ball[128:192]
    bpack = jnp.concatenate(
        [b123, ball[256:320], ball[320:384], ball[384:448]], axis=0)
    apack = jnp.concatenate(
        [aall[64:128], aall[128:192], aall[256:320], aall[320:384],
         aall[384:448]], axis=0)
    w0a1 = jnp.concatenate([W0, aall[0:64]], axis=1).astype(jnp.bfloat16)
    a4b4 = jnp.concatenate([aall[192:256], ball[192:256]],
                           axis=1).astype(jnp.bfloat16)

    adj = _build_adj(A[1], A[0])
    adjb = adj.astype(jnp.bfloat16)
    xsum, xlsum, xdiag, xps = _run_stats(X, W0, b0)
    return _run_main(X, adjb, w0a1, b0, a4b4, apack, bpack, xsum, xlsum,
                     xdiag, xps, bvec, bdiag)
